# baseline (speedup 1.0000x reference)
import functools

import jax
import jax.numpy as jnp
from jax import lax
from jax.experimental import pallas as pl
from jax.experimental.pallas import tpu as pltpu
from jax.experimental.pallas import tpu_sc as plsc

B, n, d = 32, 64, 128
E_PER = 512
L, DEPTH = 4, 2
NN = n * n
S = 2 * E_PER + n
S_PAD = 1152
NJ = S_PAD // 128
CG = 16
DC = 16
NDC = d // DC


def _encoder_kernel(pe_idx_ref, edge_idx_ref, pe_val_ref, edge_val_ref,
                    node_val_ref, Wpe_ref, bpe_ref, We_ref, be_ref,
                    Wn_ref, bn_ref, v_ref, ridx_ref):
    f32 = jnp.float32
    pe_idx = pe_idx_ref[0]
    edge_idx = edge_idx_ref[0]
    r_pe = ((pe_idx[:, 0:1] & (n - 1)) << 6) | (pe_idx[:, 1:2] & (n - 1))
    r_edge = ((edge_idx[:, 0:1] & (n - 1)) << 6) | (edge_idx[:, 1:2] & (n - 1))
    r_node = (n + 1) * jax.lax.broadcasted_iota(jnp.int32, (n, 1), 0)
    pad = jnp.zeros((S_PAD - S, 1), jnp.int32)
    r = jnp.concatenate([r_pe, r_edge, r_node, pad], axis=0)
    ridx_ref[0] = r + (pl.program_id(0) // 2) * NN

    enc_pe = jnp.dot(pe_val_ref[0], Wpe_ref[...],
                     preferred_element_type=f32) + bpe_ref[...]
    enc_edge = jnp.dot(edge_val_ref[0], We_ref[...],
                       preferred_element_type=f32) + be_ref[...]
    nv = jnp.dot(node_val_ref[0], Wn_ref[...],
                 preferred_element_type=f32) + bn_ref[...]
    v_ref[0] = jnp.concatenate(
        [enc_pe, enc_edge, nv, jnp.zeros((S_PAD - S, d), f32)], axis=0)


def _sc_scatter_kernel(v_hbm, ridx_hbm, zeros_hbm, h0_hbm,
                       rows_v, ridx_v, acc_sh):
    c = lax.axis_index("c")
    s = lax.axis_index("s")
    g = s * 2 + c
    base = s * NN

    pltpu.sync_copy(ridx_hbm.at[g], ridx_v)

    def chunk(cc, _):
        pltpu.sync_copy(zeros_hbm, acc_sh.at[pl.ds(base, NN)])
        pltpu.sync_copy(v_hbm.at[g, cc], rows_v)
        for j in range(NJ):
            pltpu.sync_copy(rows_v.at[pl.ds(j * 128, 128)],
                            acc_sh.at[ridx_v.at[j]], add=True)
        pltpu.sync_copy(acc_sh.at[pl.ds(base, NN)], h0_hbm.at[g, cc])
        return 0
    lax.fori_loop(0, NDC, chunk, 0)


def _dense_kernel(h0_ref, W1T_ref, b1_ref, W2T_ref, b2_ref, W3T_ref, b3_ref,
                  z_ref, H_ref, m1t_ref, m2t_ref, Mt_ref):
    f32 = jnp.float32
    H_ref[...] = jnp.concatenate(
        [h0_ref[0, cc].T for cc in range(NDC)], axis=0)

    def layer(l, _):
        x = H_ref[...]
        m1 = x.astype(jnp.bfloat16)
        m2 = m1
        for t in range(DEPTH):
            m1 = jax.nn.relu(jnp.dot(W1T_ref[l, t].astype(jnp.bfloat16), m1,
                                     preferred_element_type=f32)
                             + b1_ref[l, t]).astype(jnp.bfloat16)
            m2 = jax.nn.relu(jnp.dot(W2T_ref[l, t].astype(jnp.bfloat16), m2,
                                     preferred_element_type=f32)
                             + b2_ref[l, t]).astype(jnp.bfloat16)
        m1t_ref[...] = m1.reshape(d, n, n)
        m2t_ref[...] = m2.reshape(d, n, n)

        def cgroup(g, _):
            a = m1t_ref[pl.ds(g * CG, CG)]
            b = m2t_ref[pl.ds(g * CG, CG)]
            Mt_ref[pl.ds(g * CG, CG)] = jax.lax.dot_general(
                a, b, dimension_numbers=(((2,), (1,)), ((0,), (0,))),
                preferred_element_type=f32).astype(jnp.bfloat16)
            return 0
        jax.lax.fori_loop(0, d // CG, cgroup, 0)

        H_ref[...] = jax.nn.relu(
            jnp.dot(W3T_ref[l].astype(jnp.bfloat16),
                    Mt_ref[...].reshape(d, NN),
                    preferred_element_type=f32)
            + b3_ref[l]) + x
        return 0
    lax.fori_loop(0, L, layer, 0)

    p = jax.lax.broadcasted_iota(jnp.int32, (NN, 2), 0)
    sel = jax.lax.broadcasted_iota(jnp.int32, (NN, 2), 1)
    Ssel = jnp.where((sel == 1) | (p % (n + 1) == 0), 1.0, 0.0).astype(f32)
    sums = jnp.dot(H_ref[...], Ssel, preferred_element_type=f32)
    diag_sum = sums[:, 0:1]
    z_diag = diag_sum * (1.0 / n)
    z_off = (sums[:, 1:2] - diag_sum) * (1.0 / (NN - n))
    z_ref[0] = jnp.concatenate([z_diag.T, z_off.T], axis=1)


def _decoder_kernel(z_ref, Wd_ref, bd_ref, out_ref):
    out_ref[...] = jnp.dot(z_ref[...], Wd_ref[...],
                           preferred_element_type=jnp.float32) + bd_ref[...]


def kernel(batch_full_index, batch_pe_index, batch_pe_val, batch_edge_index,
           batch_edge_val, batch_eye_index, batch_node_val, total_num_nodes,
           Wpe, bpe, We, be, Wn, bn, W1, b1, W2, b2, W3, b3, Wd, bd):
    f32 = jnp.float32
    pe_idx = batch_pe_index.reshape(2, B, E_PER).transpose(1, 2, 0)
    edge_idx = batch_edge_index.reshape(2, B, E_PER).transpose(1, 2, 0)
    pe_val = batch_pe_val.reshape(B, E_PER, -1)
    edge_val = batch_edge_val.reshape(B, E_PER, -1)
    node_val = batch_node_val.reshape(B, n, -1)

    rep = lambda shape: pl.BlockSpec(shape, lambda b: (0,) * len(shape))

    v_rows, ridx3 = pl.pallas_call(
        _encoder_kernel,
        grid=(B,),
        in_specs=[
            pl.BlockSpec((1, E_PER, 2), lambda b: (b, 0, 0)),
            pl.BlockSpec((1, E_PER, 2), lambda b: (b, 0, 0)),
            pl.BlockSpec((1, E_PER, pe_val.shape[-1]), lambda b: (b, 0, 0)),
            pl.BlockSpec((1, E_PER, edge_val.shape[-1]), lambda b: (b, 0, 0)),
            pl.BlockSpec((1, n, node_val.shape[-1]), lambda b: (b, 0, 0)),
            rep(Wpe.shape), rep((1, d)),
            rep(We.shape), rep((1, d)),
            rep(Wn.shape), rep((1, d)),
        ],
        out_specs=[pl.BlockSpec((1, S_PAD, d), lambda b: (b, 0, 0)),
                   pl.BlockSpec((1, S_PAD, 1), lambda b: (b, 0, 0))],
        out_shape=[jax.ShapeDtypeStruct((B, S_PAD, d), f32),
                   jax.ShapeDtypeStruct((B, S_PAD, 1), jnp.int32)],
    )(pe_idx, edge_idx, pe_val, edge_val, node_val,
      Wpe, bpe.reshape(1, d), We, be.reshape(1, d), Wn, bn.reshape(1, d))

    v6 = v_rows.reshape(B, S_PAD, NDC, DC).transpose(0, 2, 1, 3)

    mesh = plsc.VectorSubcoreMesh(core_axis_name="c", subcore_axis_name="s")
    h0 = functools.partial(
        pl.kernel, mesh=mesh,
        compiler_params=pltpu.CompilerParams(use_tc_tiling_on_sc=False),
        out_type=jax.ShapeDtypeStruct((B, NDC, NN, DC), f32),
        scratch_types=[
            pltpu.VMEM((S_PAD, DC), f32),
            pltpu.VMEM((NJ, 128), jnp.int32),
            pltpu.VMEM_SHARED((16 * NN, DC), f32),
        ],
    )(_sc_scatter_kernel)(v6, ridx3.reshape(B, NJ, 128),
                          jnp.zeros((NN, DC), f32))

    z = pl.pallas_call(
        _dense_kernel,
        grid=(B,),
        in_specs=[
            pl.BlockSpec((1, NDC, NN, DC), lambda b: (b, 0, 0, 0)),
            rep((L, DEPTH, d, d)), rep((L, DEPTH, d, 1)),
            rep((L, DEPTH, d, d)), rep((L, DEPTH, d, 1)),
            rep((L, d, d)), rep((L, d, 1)),
        ],
        out_specs=pl.BlockSpec((1, 1, 2 * d), lambda b: (b, 0, 0)),
        out_shape=jax.ShapeDtypeStruct((B, 1, 2 * d), f32),
        scratch_shapes=[pltpu.VMEM((d, NN), f32),
                        pltpu.VMEM((d, n, n), jnp.bfloat16),
                        pltpu.VMEM((d, n, n), jnp.bfloat16),
                        pltpu.VMEM((d, n, n), jnp.bfloat16)],
    )(h0,
      W1.transpose(0, 1, 3, 2), b1.reshape(L, DEPTH, d, 1),
      W2.transpose(0, 1, 3, 2), b2.reshape(L, DEPTH, d, 1),
      W3.transpose(0, 2, 1) * (1.0 / n), b3.reshape(L, d, 1))

    out = pl.pallas_call(
        _decoder_kernel,
        out_shape=jax.ShapeDtypeStruct((B, 1), f32),
    )(z.reshape(B, 2 * d), Wd, bd.reshape(1, 1))
    return out


# speedup vs baseline: 1.6752x; 1.6752x over previous
import jax
import jax.numpy as jnp
from jax.experimental import pallas as pl
from jax.experimental.pallas import tpu as pltpu

B, n, d = 32, 64, 128
E_PER = 512
L, DEPTH = 4, 2
NN = n * n
CH = 512
CG = 16


def _gnn_block_kernel(pe_idx_ref, edge_idx_ref, pe_val_ref, edge_val_ref,
                      node_val_ref, WpeT_ref, bpe_ref, WeT_ref, be_ref,
                      WnT_ref, bn_ref, W1T_ref, b1_ref, W2T_ref, b2_ref,
                      W3T_ref, b3_ref, Ssel_ref, z_ref, H_ref, m1t_ref,
                      m2t_ref, Mt_ref):
    f32 = jnp.float32

    pe_idx = pe_idx_ref[0]
    edge_idx = edge_idx_ref[0]
    r_pe = ((pe_idx[:, 0:1] & (n - 1)) << 6) | (pe_idx[:, 1:2] & (n - 1))
    r_edge = ((edge_idx[:, 0:1] & (n - 1)) << 6) | (edge_idx[:, 1:2] & (n - 1))
    r = jnp.concatenate([r_pe, r_edge], axis=0)

    enc_pe = jnp.dot(WpeT_ref[...], pe_val_ref[0],
                     preferred_element_type=f32) + bpe_ref[...]
    enc_edge = jnp.dot(WeT_ref[...], edge_val_ref[0],
                       preferred_element_type=f32) + be_ref[...]
    nvT = jnp.dot(WnT_ref[...], node_val_ref[0],
                  preferred_element_type=f32) + bn_ref[...]
    r_node = (n + 1) * jax.lax.broadcasted_iota(jnp.int32, (n, 1), 0)
    VT = jnp.concatenate([enc_pe, enc_edge, nvT], axis=1)
    r = jnp.concatenate([r, r_node], axis=0)

    VTb = VT.astype(jnp.bfloat16)
    def scatter_chunk(c, _):
        cols = c * CH + jax.lax.broadcasted_iota(jnp.int32, (1, CH), 1)
        oh = (r == cols).astype(jnp.bfloat16)
        H_ref[:, pl.ds(c * CH, CH)] = jnp.dot(VTb, oh, preferred_element_type=f32)
        return 0
    jax.lax.fori_loop(0, NN // CH, scatter_chunk, 0)

    def layer(l, _):
        x = H_ref[...]
        m1 = x.astype(jnp.bfloat16)
        m2 = m1
        for t in range(DEPTH):
            m1 = jax.nn.relu(jnp.dot(W1T_ref[l, t].astype(jnp.bfloat16), m1,
                                     preferred_element_type=f32)
                             + b1_ref[l, t]).astype(jnp.bfloat16)
            m2 = jax.nn.relu(jnp.dot(W2T_ref[l, t].astype(jnp.bfloat16), m2,
                                     preferred_element_type=f32)
                             + b2_ref[l, t]).astype(jnp.bfloat16)
        m1t_ref[...] = m1.reshape(d, n, n)
        m2t_ref[...] = m2.reshape(d, n, n)

        def cgroup(g, _):
            a = m1t_ref[pl.ds(g * CG, CG)]
            b = m2t_ref[pl.ds(g * CG, CG)]
            Mt_ref[pl.ds(g * CG, CG)] = jax.lax.dot_general(
                a, b, dimension_numbers=(((2,), (1,)), ((0,), (0,))),
                preferred_element_type=f32).astype(jnp.bfloat16)
            return 0
        jax.lax.fori_loop(0, d // CG, cgroup, 0)

        H_ref[...] = jax.nn.relu(
            jnp.dot(W3T_ref[l].astype(jnp.bfloat16),
                    Mt_ref[...].reshape(d, NN),
                    preferred_element_type=f32)
            + b3_ref[l]) + x
        return 0
    jax.lax.fori_loop(0, L, layer, 0)

    sums = jnp.dot(H_ref[...], Ssel_ref[...],
                   preferred_element_type=f32)
    diag_sum = sums[:, 0:1]
    z_diag = diag_sum * (1.0 / n)
    z_off = (sums[:, 1:2] - diag_sum) * (1.0 / (NN - n))
    z_ref[0] = jnp.concatenate([z_diag.T, z_off.T], axis=1)


def _decoder_kernel(z_ref, Wd_ref, bd_ref, out_ref):
    out_ref[...] = jnp.dot(z_ref[...], Wd_ref[...],
                           preferred_element_type=jnp.float32) + bd_ref[...]


def kernel(batch_full_index, batch_pe_index, batch_pe_val, batch_edge_index,
           batch_edge_val, batch_eye_index, batch_node_val, total_num_nodes,
           Wpe, bpe, We, be, Wn, bn, W1, b1, W2, b2, W3, b3, Wd, bd):
    f32 = jnp.float32
    pe_idx = batch_pe_index.reshape(2, B, E_PER).transpose(1, 2, 0)
    edge_idx = batch_edge_index.reshape(2, B, E_PER).transpose(1, 2, 0)
    pe_val = batch_pe_val.reshape(B, E_PER, -1).transpose(0, 2, 1)
    edge_val = batch_edge_val.reshape(B, E_PER, -1).transpose(0, 2, 1)
    node_val = batch_node_val.reshape(B, n, -1).transpose(0, 2, 1)

    rep = lambda shape: pl.BlockSpec(shape, lambda b: (0,) * len(shape))

    z = pl.pallas_call(
        _gnn_block_kernel,
        grid=(B,),
        in_specs=[
            pl.BlockSpec((1, E_PER, 2), lambda b: (b, 0, 0)),
            pl.BlockSpec((1, E_PER, 2), lambda b: (b, 0, 0)),
            pl.BlockSpec((1, pe_val.shape[1], E_PER), lambda b: (b, 0, 0)),
            pl.BlockSpec((1, edge_val.shape[1], E_PER), lambda b: (b, 0, 0)),
            pl.BlockSpec((1, node_val.shape[1], n), lambda b: (b, 0, 0)),
            rep((d, Wpe.shape[0])), rep((d, 1)),
            rep((d, We.shape[0])), rep((d, 1)),
            rep((d, Wn.shape[0])), rep((d, 1)),
            rep((L, DEPTH, d, d)), rep((L, DEPTH, d, 1)),
            rep((L, DEPTH, d, d)), rep((L, DEPTH, d, 1)),
            rep((L, d, d)), rep((L, d, 1)),
            rep((NN, 2)),
        ],
        out_specs=pl.BlockSpec((1, 1, 2 * d), lambda b: (b, 0, 0)),
        out_shape=jax.ShapeDtypeStruct((B, 1, 2 * d), f32),
        scratch_shapes=[pltpu.VMEM((d, NN), f32),
                        pltpu.VMEM((d, n, n), jnp.bfloat16),
                        pltpu.VMEM((d, n, n), jnp.bfloat16),
                        pltpu.VMEM((d, n, n), jnp.bfloat16)],
    )(pe_idx, edge_idx, pe_val, edge_val, node_val,
      Wpe.T, bpe.reshape(d, 1), We.T, be.reshape(d, 1), Wn.T, bn.reshape(d, 1),
      W1.transpose(0, 1, 3, 2), b1.reshape(L, DEPTH, d, 1),
      W2.transpose(0, 1, 3, 2), b2.reshape(L, DEPTH, d, 1),
      W3.transpose(0, 2, 1) * (1.0 / n), b3.reshape(L, d, 1),
      jnp.concatenate(
          [(jnp.arange(NN, dtype=jnp.int32) % (n + 1) == 0
            ).astype(f32).reshape(NN, 1),
           jnp.ones((NN, 1), f32)], axis=1))

    out = pl.pallas_call(
        _decoder_kernel,
        out_shape=jax.ShapeDtypeStruct((B, 1), f32),
    )(z.reshape(B, 2 * d), Wd, bd.reshape(1, 1))
    return out


# speedup vs baseline: 2.3261x; 1.3885x over previous
import jax
import jax.numpy as jnp
from jax.experimental import pallas as pl
from jax.experimental.pallas import tpu as pltpu

B, n, d = 32, 64, 128
E_PER = 512
L, DEPTH = 4, 2
NN = n * n
CH = 512
CG = 16


def _gnn_block_kernel(pe_idx_ref, edge_idx_ref, pe_val_ref, edge_val_ref,
                      node_val_ref, WpeT_ref, bpe_ref, WeT_ref, be_ref,
                      WnT_ref, bn_ref, W1T_ref, b1_ref, W2T_ref, b2_ref,
                      W3T_ref, b3_ref, Ssel_ref, z_ref, H_ref):
    f32 = jnp.float32

    pe_idx = pe_idx_ref[0]
    edge_idx = edge_idx_ref[0]
    r_pe = ((pe_idx[:, 0:1] & (n - 1)) << 6) | (pe_idx[:, 1:2] & (n - 1))
    r_edge = ((edge_idx[:, 0:1] & (n - 1)) << 6) | (edge_idx[:, 1:2] & (n - 1))
    r = jnp.concatenate([r_pe, r_edge], axis=0)

    enc_pe = jnp.dot(WpeT_ref[...], pe_val_ref[0],
                     preferred_element_type=f32) + bpe_ref[...]
    enc_edge = jnp.dot(WeT_ref[...], edge_val_ref[0],
                       preferred_element_type=f32) + be_ref[...]
    nvT = jnp.dot(WnT_ref[...], node_val_ref[0],
                  preferred_element_type=f32) + bn_ref[...]
    r_node = (n + 1) * jax.lax.broadcasted_iota(jnp.int32, (n, 1), 0)
    VT = jnp.concatenate([enc_pe, enc_edge, nvT], axis=1)
    r = jnp.concatenate([r, r_node], axis=0)

    VTb = VT.astype(jnp.bfloat16)
    def scatter_chunk(c, _):
        cols = c * CH + jax.lax.broadcasted_iota(jnp.int32, (1, CH), 1)
        oh = (r == cols).astype(jnp.bfloat16)
        H_ref[:, pl.ds(c * CH, CH)] = jnp.dot(VTb, oh, preferred_element_type=f32)
        return 0
    jax.lax.fori_loop(0, NN // CH, scatter_chunk, 0)

    def layer(l, _):
        x = H_ref[...]
        m1 = x.astype(jnp.bfloat16)
        m2 = m1
        for t in range(DEPTH):
            m1 = jax.nn.relu(jnp.dot(W1T_ref[l, t].astype(jnp.bfloat16), m1,
                                     preferred_element_type=f32)
                             + b1_ref[l, t]).astype(jnp.bfloat16)
            m2 = jax.nn.relu(jnp.dot(W2T_ref[l, t].astype(jnp.bfloat16), m2,
                                     preferred_element_type=f32)
                             + b2_ref[l, t]).astype(jnp.bfloat16)
        Mt = jax.lax.dot_general(
            m1.reshape(d, n, n), m2.reshape(d, n, n),
            dimension_numbers=(((2,), (1,)), ((0,), (0,))),
            preferred_element_type=f32).astype(jnp.bfloat16)

        H_ref[...] = jax.nn.relu(
            jnp.dot(W3T_ref[l].astype(jnp.bfloat16),
                    Mt.reshape(d, NN),
                    preferred_element_type=f32)
            + b3_ref[l]) + x
        return 0
    jax.lax.fori_loop(0, L, layer, 0)

    sums = jnp.dot(H_ref[...], Ssel_ref[...],
                   preferred_element_type=f32)
    diag_sum = sums[:, 0:1]
    z_diag = diag_sum * (1.0 / n)
    z_off = (sums[:, 1:2] - diag_sum) * (1.0 / (NN - n))
    z_ref[0] = jnp.concatenate([z_diag.T, z_off.T], axis=1)


def _decoder_kernel(z_ref, Wd_ref, bd_ref, out_ref):
    out_ref[...] = jnp.dot(z_ref[...], Wd_ref[...],
                           preferred_element_type=jnp.float32) + bd_ref[...]


def kernel(batch_full_index, batch_pe_index, batch_pe_val, batch_edge_index,
           batch_edge_val, batch_eye_index, batch_node_val, total_num_nodes,
           Wpe, bpe, We, be, Wn, bn, W1, b1, W2, b2, W3, b3, Wd, bd):
    f32 = jnp.float32
    pe_idx = batch_pe_index.reshape(2, B, E_PER).transpose(1, 2, 0)
    edge_idx = batch_edge_index.reshape(2, B, E_PER).transpose(1, 2, 0)
    pe_val = batch_pe_val.reshape(B, E_PER, -1).transpose(0, 2, 1)
    edge_val = batch_edge_val.reshape(B, E_PER, -1).transpose(0, 2, 1)
    node_val = batch_node_val.reshape(B, n, -1).transpose(0, 2, 1)

    rep = lambda shape: pl.BlockSpec(shape, lambda b: (0,) * len(shape))

    z = pl.pallas_call(
        _gnn_block_kernel,
        grid=(B,),
        in_specs=[
            pl.BlockSpec((1, E_PER, 2), lambda b: (b, 0, 0)),
            pl.BlockSpec((1, E_PER, 2), lambda b: (b, 0, 0)),
            pl.BlockSpec((1, pe_val.shape[1], E_PER), lambda b: (b, 0, 0)),
            pl.BlockSpec((1, edge_val.shape[1], E_PER), lambda b: (b, 0, 0)),
            pl.BlockSpec((1, node_val.shape[1], n), lambda b: (b, 0, 0)),
            rep((d, Wpe.shape[0])), rep((d, 1)),
            rep((d, We.shape[0])), rep((d, 1)),
            rep((d, Wn.shape[0])), rep((d, 1)),
            rep((L, DEPTH, d, d)), rep((L, DEPTH, d, 1)),
            rep((L, DEPTH, d, d)), rep((L, DEPTH, d, 1)),
            rep((L, d, d)), rep((L, d, 1)),
            rep((NN, 2)),
        ],
        out_specs=pl.BlockSpec((1, 1, 2 * d), lambda b: (b, 0, 0)),
        out_shape=jax.ShapeDtypeStruct((B, 1, 2 * d), f32),
        scratch_shapes=[pltpu.VMEM((d, NN), f32)],
    )(pe_idx, edge_idx, pe_val, edge_val, node_val,
      Wpe.T, bpe.reshape(d, 1), We.T, be.reshape(d, 1), Wn.T, bn.reshape(d, 1),
      W1.transpose(0, 1, 3, 2), b1.reshape(L, DEPTH, d, 1),
      W2.transpose(0, 1, 3, 2), b2.reshape(L, DEPTH, d, 1),
      W3.transpose(0, 2, 1) * (1.0 / n), b3.reshape(L, d, 1),
      jnp.concatenate(
          [(jnp.arange(NN, dtype=jnp.int32) % (n + 1) == 0
            ).astype(f32).reshape(NN, 1),
           jnp.ones((NN, 1), f32)], axis=1))

    out = pl.pallas_call(
        _decoder_kernel,
        out_shape=jax.ShapeDtypeStruct((B, 1), f32),
    )(z.reshape(B, 2 * d), Wd, bd.reshape(1, 1))
    return out


# speedup vs baseline: 2.5832x; 1.1105x over previous
import jax
import jax.numpy as jnp
from jax.experimental import pallas as pl
from jax.experimental.pallas import tpu as pltpu

B, n, d = 32, 64, 128
E_PER = 512
L, DEPTH = 4, 2
NN = n * n
CH = 512
CG = 16


def _gnn_block_kernel(pe_idx_ref, edge_idx_ref, pe_val_ref, edge_val_ref,
                      node_val_ref, WpeT_ref, bpe_ref, WeT_ref, be_ref,
                      WnT_ref, bn_ref, W1T_ref, b1_ref, W2T_ref, b2_ref,
                      W3T_ref, b3_ref, Ssel_ref, z_ref, H_ref):
    f32 = jnp.float32

    pe_idx = pe_idx_ref[0]
    edge_idx = edge_idx_ref[0]
    r_pe = ((pe_idx[:, 0:1] & (n - 1)) << 6) | (pe_idx[:, 1:2] & (n - 1))
    r_edge = ((edge_idx[:, 0:1] & (n - 1)) << 6) | (edge_idx[:, 1:2] & (n - 1))
    r = jnp.concatenate([r_pe, r_edge], axis=0)

    enc_pe = jnp.dot(WpeT_ref[...], pe_val_ref[0],
                     preferred_element_type=f32) + bpe_ref[...]
    enc_edge = jnp.dot(WeT_ref[...], edge_val_ref[0],
                       preferred_element_type=f32) + be_ref[...]
    nvT = jnp.dot(WnT_ref[...], node_val_ref[0],
                  preferred_element_type=f32) + bn_ref[...]
    r_node = (n + 1) * jax.lax.broadcasted_iota(jnp.int32, (n, 1), 0)
    VT = jnp.concatenate([enc_pe, enc_edge, nvT], axis=1)
    r = jnp.concatenate([r, r_node], axis=0)

    cols = jax.lax.broadcasted_iota(jnp.int32, (1, NN), 1)
    oh = (r == cols).astype(jnp.bfloat16)
    H_ref[...] = jnp.dot(VT.astype(jnp.bfloat16), oh,
                         preferred_element_type=f32)

    def layer(l, _):
        x = H_ref[...]
        m1 = x.astype(jnp.bfloat16)
        m2 = m1
        for t in range(DEPTH):
            m1 = jax.nn.relu(jnp.dot(W1T_ref[l, t].astype(jnp.bfloat16), m1,
                                     preferred_element_type=f32)
                             + b1_ref[l, t]).astype(jnp.bfloat16)
            m2 = jax.nn.relu(jnp.dot(W2T_ref[l, t].astype(jnp.bfloat16), m2,
                                     preferred_element_type=f32)
                             + b2_ref[l, t]).astype(jnp.bfloat16)
        Mt = jax.lax.dot_general(
            m1.reshape(d, n, n), m2.reshape(d, n, n),
            dimension_numbers=(((2,), (1,)), ((0,), (0,))),
            preferred_element_type=f32).astype(jnp.bfloat16)

        H_ref[...] = jax.nn.relu(
            jnp.dot(W3T_ref[l].astype(jnp.bfloat16),
                    Mt.reshape(d, NN),
                    preferred_element_type=f32)
            + b3_ref[l]) + x
        return 0
    jax.lax.fori_loop(0, L, layer, 0)

    sums = jnp.dot(H_ref[...], Ssel_ref[...],
                   preferred_element_type=f32)
    diag_sum = sums[:, 0:1]
    z_diag = diag_sum * (1.0 / n)
    z_off = (sums[:, 1:2] - diag_sum) * (1.0 / (NN - n))
    z_ref[0] = jnp.concatenate([z_diag.T, z_off.T], axis=1)


def _decoder_kernel(z_ref, Wd_ref, bd_ref, out_ref):
    out_ref[...] = jnp.dot(z_ref[...], Wd_ref[...],
                           preferred_element_type=jnp.float32) + bd_ref[...]


def kernel(batch_full_index, batch_pe_index, batch_pe_val, batch_edge_index,
           batch_edge_val, batch_eye_index, batch_node_val, total_num_nodes,
           Wpe, bpe, We, be, Wn, bn, W1, b1, W2, b2, W3, b3, Wd, bd):
    f32 = jnp.float32
    pe_idx = batch_pe_index.reshape(2, B, E_PER).transpose(1, 2, 0)
    edge_idx = batch_edge_index.reshape(2, B, E_PER).transpose(1, 2, 0)
    pe_val = batch_pe_val.reshape(B, E_PER, -1).transpose(0, 2, 1)
    edge_val = batch_edge_val.reshape(B, E_PER, -1).transpose(0, 2, 1)
    node_val = batch_node_val.reshape(B, n, -1).transpose(0, 2, 1)

    rep = lambda shape: pl.BlockSpec(shape, lambda b: (0,) * len(shape))

    z = pl.pallas_call(
        _gnn_block_kernel,
        grid=(B,),
        in_specs=[
            pl.BlockSpec((1, E_PER, 2), lambda b: (b, 0, 0)),
            pl.BlockSpec((1, E_PER, 2), lambda b: (b, 0, 0)),
            pl.BlockSpec((1, pe_val.shape[1], E_PER), lambda b: (b, 0, 0)),
            pl.BlockSpec((1, edge_val.shape[1], E_PER), lambda b: (b, 0, 0)),
            pl.BlockSpec((1, node_val.shape[1], n), lambda b: (b, 0, 0)),
            rep((d, Wpe.shape[0])), rep((d, 1)),
            rep((d, We.shape[0])), rep((d, 1)),
            rep((d, Wn.shape[0])), rep((d, 1)),
            rep((L, DEPTH, d, d)), rep((L, DEPTH, d, 1)),
            rep((L, DEPTH, d, d)), rep((L, DEPTH, d, 1)),
            rep((L, d, d)), rep((L, d, 1)),
            rep((NN, 2)),
        ],
        out_specs=pl.BlockSpec((1, 1, 2 * d), lambda b: (b, 0, 0)),
        out_shape=jax.ShapeDtypeStruct((B, 1, 2 * d), f32),
        scratch_shapes=[pltpu.VMEM((d, NN), f32)],
    )(pe_idx, edge_idx, pe_val, edge_val, node_val,
      Wpe.T, bpe.reshape(d, 1), We.T, be.reshape(d, 1), Wn.T, bn.reshape(d, 1),
      W1.transpose(0, 1, 3, 2), b1.reshape(L, DEPTH, d, 1),
      W2.transpose(0, 1, 3, 2), b2.reshape(L, DEPTH, d, 1),
      W3.transpose(0, 2, 1) * (1.0 / n), b3.reshape(L, d, 1),
      jnp.concatenate(
          [(jnp.arange(NN, dtype=jnp.int32) % (n + 1) == 0
            ).astype(f32).reshape(NN, 1),
           jnp.ones((NN, 1), f32)], axis=1))

    out = pl.pallas_call(
        _decoder_kernel,
        out_shape=jax.ShapeDtypeStruct((B, 1), f32),
    )(z.reshape(B, 2 * d), Wd, bd.reshape(1, 1))
    return out
